# baseline (speedup 1.0000x reference)
import functools

import jax
import jax.numpy as jnp
from jax import lax
from jax.experimental import pallas as pl
from jax.experimental.pallas import tpu as pltpu
from jax.experimental.pallas import tpu_sc as plsc

DIM = 32
CHUNK = 1024
L = 16


@functools.cache
def _make_gather(H: int, NB: int, n_rows: int):
    info = plsc.get_sparse_core_info()
    NC, NS = info.num_cores, info.num_subcores
    NW = NC * NS
    tasks_per_h = NB // CHUNK
    total_tasks = H * tasks_per_h
    assert total_tasks % NW == 0
    tasks_pw = total_tasks // NW

    mesh = plsc.VectorSubcoreMesh(core_axis_name="c", subcore_axis_name="s")

    @functools.partial(
        pl.kernel,
        mesh=mesh,
        out_type=jax.ShapeDtypeStruct((H * DIM, NB), jnp.float32),
        scratch_types=[
            [pltpu.VMEM((CHUNK,), jnp.int32)] * 2,
            [pltpu.VMEM((CHUNK, DIM), jnp.float32)] * 2,
            pltpu.VMEM((DIM, CHUNK), jnp.float32),
            [pltpu.SemaphoreType.DMA] * 2,
            [pltpu.SemaphoreType.DMA] * 2,
            pltpu.SemaphoreType.DMA,
        ],
        compiler_params=pltpu.CompilerParams(
            use_tc_tiling_on_sc=False, needs_layout_passes=False
        ),
    )
    def gather_kernel(table_hbm, idx_hbm, out_hbm, idxs, rows, tr, isems, gsems, osem):
        wid = lax.axis_index("s") * NC + lax.axis_index("c")
        t0 = wid * tasks_pw

        def start_idx(k, s):
            pltpu.async_copy(
                idx_hbm.at[pl.ds((t0 + k) * CHUNK, CHUNK)], idxs[s], isems[s]
            )

        def wait_idx(s):
            pltpu.make_async_copy(
                idx_hbm.at[pl.ds(0, CHUNK)], idxs[s], isems[s]
            ).wait()

        def start_gather(s):
            pltpu.async_copy(table_hbm.at[idxs[s]], rows[s], gsems[s])

        def wait_gather(s):
            pltpu.make_async_copy(
                table_hbm.at[pl.ds(0, CHUNK)], rows[s], gsems[s]
            ).wait()

        def start_store(k):
            t = t0 + k
            h = t // tasks_per_h
            b0 = (t % tasks_per_h) * CHUNK
            pltpu.async_copy(
                tr, out_hbm.at[pl.ds(h * DIM, DIM), pl.ds(b0, CHUNK)], osem
            )

        def wait_store():
            pltpu.make_async_copy(
                tr, out_hbm.at[pl.ds(0, DIM), pl.ds(0, CHUNK)], osem
            ).wait()

        def transpose(s):
            rows_ref = rows[s]

            def jloop(jb, _):
                j0 = jb * L
                ridx = j0 + lax.iota(jnp.int32, L)
                for d in range(DIM):
                    col = jnp.full((L,), d, jnp.int32)
                    v = plsc.load_gather(rows_ref, [ridx, col])
                    tr[d, pl.ds(j0, L)] = v
                return ()

            lax.fori_loop(0, CHUNK // L, jloop, ())

        start_idx(0, 0)
        if tasks_pw > 1:
            start_idx(1, 1)
        wait_idx(0)
        start_gather(0)

        for k in range(tasks_pw):
            s = k % 2
            if k + 1 < tasks_pw:
                wait_idx(1 - s)
            wait_gather(s)
            if k + 1 < tasks_pw:
                start_gather(1 - s)
            if k + 2 < tasks_pw:
                start_idx(k + 2, s)
            if k >= 1:
                wait_store()
            transpose(s)
            start_store(k)

        wait_store()

    return gather_kernel


def kernel(instance, concept, table):
    batch, hist = instance.shape
    idx_hm = jnp.transpose(instance).reshape(-1).astype(jnp.int32)
    out = _make_gather(hist, batch, table.shape[0])(table, idx_hm)
    return jnp.transpose(out.reshape(hist, DIM, batch), (2, 0, 1))


# speedup vs baseline: 1.6571x; 1.6571x over previous
import functools

import jax
import jax.numpy as jnp
from jax import lax
from jax.experimental import pallas as pl
from jax.experimental.pallas import tpu as pltpu
from jax.experimental.pallas import tpu_sc as plsc

DIM = 32
CHUNK = 1024
L = 16


@functools.cache
def _make_gather(H: int, NB: int, n_rows: int):
    info = plsc.get_sparse_core_info()
    NC, NS = info.num_cores, info.num_subcores
    NW = NC * NS
    tasks_per_h = NB // CHUNK
    total_tasks = H * tasks_per_h
    assert total_tasks % NW == 0
    tasks_pw = total_tasks // NW

    mesh = plsc.VectorSubcoreMesh(core_axis_name="c", subcore_axis_name="s")

    @functools.partial(
        pl.kernel,
        mesh=mesh,
        out_type=jax.ShapeDtypeStruct((H * DIM, NB), jnp.float32),
        scratch_types=[
            [pltpu.VMEM((CHUNK,), jnp.int32)] * 2,
            [pltpu.VMEM((CHUNK, DIM), jnp.float32)] * 2,
            pltpu.VMEM((DIM, CHUNK + 1), jnp.float32),
            [pltpu.SemaphoreType.DMA] * 2,
            [pltpu.SemaphoreType.DMA] * 2,
            pltpu.SemaphoreType.DMA,
        ],
        compiler_params=pltpu.CompilerParams(
            use_tc_tiling_on_sc=False, needs_layout_passes=False
        ),
    )
    def gather_kernel(table_hbm, idx_hbm, out_hbm, idxs, rows, tr, isems, gsems, osem):
        wid = lax.axis_index("s") * NC + lax.axis_index("c")
        t0 = wid * tasks_pw

        def start_idx(k, s):
            pltpu.async_copy(
                idx_hbm.at[pl.ds((t0 + k) * CHUNK, CHUNK)], idxs[s], isems[s]
            )

        def wait_idx(s):
            pltpu.make_async_copy(
                idx_hbm.at[pl.ds(0, CHUNK)], idxs[s], isems[s]
            ).wait()

        def start_gather(s):
            pltpu.async_copy(table_hbm.at[idxs[s]], rows[s], gsems[s])

        def wait_gather(s):
            pltpu.make_async_copy(
                table_hbm.at[pl.ds(0, CHUNK)], rows[s], gsems[s]
            ).wait()

        def start_store(k):
            t = t0 + k
            h = t // tasks_per_h
            b0 = (t % tasks_per_h) * CHUNK
            pltpu.async_copy(
                tr.at[:, pl.ds(0, CHUNK)],
                out_hbm.at[pl.ds(h * DIM, DIM), pl.ds(b0, CHUNK)],
                osem,
            )

        def wait_store():
            pltpu.make_async_copy(
                tr.at[:, pl.ds(0, CHUNK)],
                out_hbm.at[pl.ds(0, DIM), pl.ds(0, CHUNK)],
                osem,
            ).wait()

        def transpose(s):
            rows_ref = rows[s]
            dlo = lax.iota(jnp.int32, L)
            dhi = dlo + L

            def jloop(jb, _):
                for u in range(4):
                    j = jb * 4 + u
                    colj = jnp.full((L,), j, jnp.int32)
                    va = rows_ref[j, pl.ds(0, L)]
                    vb = rows_ref[j, pl.ds(L, L)]
                    plsc.store_scatter(tr, [dlo, colj], va)
                    plsc.store_scatter(tr, [dhi, colj], vb)
                return ()

            lax.fori_loop(0, CHUNK // 4, jloop, ())

        start_idx(0, 0)
        if tasks_pw > 1:
            start_idx(1, 1)
        wait_idx(0)
        start_gather(0)

        for k in range(tasks_pw):
            s = k % 2
            if k + 1 < tasks_pw:
                wait_idx(1 - s)
            wait_gather(s)
            if k + 1 < tasks_pw:
                start_gather(1 - s)
            if k + 2 < tasks_pw:
                start_idx(k + 2, s)
            if k >= 1:
                wait_store()
            transpose(s)
            start_store(k)

        wait_store()

    return gather_kernel


def kernel(instance, concept, table):
    batch, hist = instance.shape
    idx_hm = jnp.transpose(instance).reshape(-1).astype(jnp.int32)
    out = _make_gather(hist, batch, table.shape[0])(table, idx_hm)
    return jnp.transpose(out.reshape(hist, DIM, batch), (2, 0, 1))
